# baseline (speedup 1.0000x reference)
import functools

import jax
import jax.numpy as jnp
from jax import lax
from jax.experimental import pallas as pl
from jax.experimental.pallas import tpu as pltpu
from jax.experimental.pallas import tpu_sc as plsc

K_TOP = 32
L = 16
NC = 2
NS = 16
NW = NC * NS
GW = 128
GP = 64
NB = 8
NEG = float("-inf")
BIG = 1 << 30


def _matmul_kernel(f1_ref, f2_ref, sim_ref, m1_ref, p1_ref, m2_ref, p2_ref):
    s = jax.lax.dot_general(
        f1_ref[...], f2_ref[...], (((0,), (0,)), ((), ())),
        preferred_element_type=jnp.float32,
    )
    sim_ref[...] = s
    nt, m = s.shape
    ng = m // GW
    s3 = s.reshape(nt, ng, GW)
    gbase = jax.lax.broadcasted_iota(jnp.int32, (nt, ng), 1) * GW
    m1 = jnp.max(s3, axis=2)
    r1 = jnp.argmax(s3, axis=2).astype(jnp.int32)
    p1 = r1 + gbase
    colg = jax.lax.broadcasted_iota(jnp.int32, (nt, ng, GW), 2)
    s3b = jnp.where(colg == r1[..., None], NEG, s3)
    m2 = jnp.max(s3b, axis=2)
    p2 = jnp.argmax(s3b, axis=2).astype(jnp.int32) + gbase
    padv = jnp.full((nt, GP - ng), NEG, jnp.float32)
    padi = jnp.full((nt, GP - ng), BIG, jnp.int32)
    m1_ref[...] = jnp.concatenate([m1, padv], axis=1)
    p1_ref[...] = jnp.concatenate([p1, padi], axis=1)
    m2_ref[...] = jnp.concatenate([m2, padv], axis=1)
    p2_ref[...] = jnp.concatenate([p2, padi], axis=1)


def _sim_pallas(f1, f2, n_tile):
    C, N = f1.shape
    M = f2.shape[1]
    return pl.pallas_call(
        _matmul_kernel,
        grid=(N // n_tile,),
        in_specs=[
            pl.BlockSpec((C, n_tile), lambda i: (0, i)),
            pl.BlockSpec((C, M), lambda i: (0, 0)),
        ],
        out_specs=[
            pl.BlockSpec((n_tile, M), lambda i: (i, 0)),
            pl.BlockSpec((n_tile, GP), lambda i: (i, 0)),
            pl.BlockSpec((n_tile, GP), lambda i: (i, 0)),
            pl.BlockSpec((n_tile, GP), lambda i: (i, 0)),
            pl.BlockSpec((n_tile, GP), lambda i: (i, 0)),
        ],
        out_shape=[
            jax.ShapeDtypeStruct((N, M), jnp.float32),
            jax.ShapeDtypeStruct((N, GP), jnp.float32),
            jax.ShapeDtypeStruct((N, GP), jnp.int32),
            jax.ShapeDtypeStruct((N, GP), jnp.float32),
            jax.ShapeDtypeStruct((N, GP), jnp.int32),
        ],
    )(f1, f2)


def _smax(x):
    return lax.sort(x)[L - 1]


def _smin(x):
    return lax.sort(x)[0]


def _sc_topk(sim, m1p, p1p, m2p, p2p):
    N, M = sim.shape
    RPW = N // NW
    NG = M // GW
    NGV = GP // L

    mesh = plsc.VectorSubcoreMesh(core_axis_name="c", subcore_axis_name="s")

    aux = lambda: pltpu.VMEM((GP + L,), jnp.float32)
    auxi = lambda: pltpu.VMEM((GP + L,), jnp.int32)

    @functools.partial(
        pl.kernel, mesh=mesh,
        compiler_params=pltpu.CompilerParams(needs_layout_passes=False),
        out_type=[jax.ShapeDtypeStruct((N * K_TOP,), jnp.float32),
                  jax.ShapeDtypeStruct((N * K_TOP,), jnp.int32)],
        scratch_types=(
            [pltpu.VMEM((M,), jnp.float32) for _ in range(NB)]
            + [aux() for _ in range(NB)]
            + [auxi() for _ in range(NB)]
            + [aux() for _ in range(NB)]
            + [auxi() for _ in range(NB)]
            + [auxi(), auxi(),
               pltpu.VMEM((RPW * K_TOP,), jnp.float32),
               pltpu.VMEM((RPW * K_TOP,), jnp.int32)]
            + [pltpu.SemaphoreType.DMA for _ in range(NB)]
        ),
    )
    def run(sim_hbm, m1_hbm, p1_hbm, m2_hbm, p2_hbm, vals_hbm, idx_hbm,
            *scr):
        rbs = scr[0:NB]
        mbs = scr[NB:2 * NB]
        pbs = scr[2 * NB:3 * NB]
        m2bs = scr[3 * NB:4 * NB]
        p2bs = scr[4 * NB:5 * NB]
        depth0, depth1, ovb, oib = scr[5 * NB:5 * NB + 4]
        sems = scr[5 * NB + 4:5 * NB + 4 + NB]
        wid = lax.axis_index("s") * NC + lax.axis_index("c")
        base_row = wid * RPW
        iota = lax.iota(jnp.int32, L)
        negv = jnp.full((L,), NEG, jnp.float32)
        bigv = jnp.full((L,), BIG, jnp.int32)
        zerov = jnp.full((L,), 0, jnp.int32)

        bufs = tuple(
            (rbs[b], mbs[b], pbs[b], m2bs[b], p2bs[b], sems[b])
            for b in range(NB))

        def fire(row, b):
            rb, mb, pb, m2b, p2b, sem = bufs[b]
            pltpu.async_copy(sim_hbm.at[row], rb, sem)
            pltpu.async_copy(m1_hbm.at[pl.ds(row * GP, GP)], mb.at[pl.ds(0, GP)], sem)
            pltpu.async_copy(p1_hbm.at[pl.ds(row * GP, GP)], pb.at[pl.ds(0, GP)], sem)
            pltpu.async_copy(m2_hbm.at[pl.ds(row * GP, GP)], m2b.at[pl.ds(0, GP)], sem)
            pltpu.async_copy(p2_hbm.at[pl.ds(row * GP, GP)], p2b.at[pl.ds(0, GP)], sem)

        def drain(row, b):
            rb, mb, pb, m2b, p2b, sem = bufs[b]
            pltpu.make_async_copy(sim_hbm.at[row], rb, sem).wait()
            pltpu.make_async_copy(m1_hbm.at[pl.ds(row * GP, GP)], mb.at[pl.ds(0, GP)], sem).wait()
            pltpu.make_async_copy(p1_hbm.at[pl.ds(row * GP, GP)], pb.at[pl.ds(0, GP)], sem).wait()
            pltpu.make_async_copy(m2_hbm.at[pl.ds(row * GP, GP)], m2b.at[pl.ds(0, GP)], sem).wait()
            pltpu.make_async_copy(p2_hbm.at[pl.ds(row * GP, GP)], p2b.at[pl.ds(0, GP)], sem).wait()

        for b in range(NB):
            fire(base_row + b, b)

        def one_step(j, st, b, dbuf):
            rb, mb, pb, m2b, p2b, _ = bufs[b]
            ov0, ov1, oi0, oi1, c0, c1, c2, c3, q0, q1, q2, q3 = st
            cur = [c0, c1, c2, c3]
            curp = [q0, q1, q2, q3]
            mv, pvl = cur[0], curp[0]
            for q in range(1, NGV):
                gt = cur[q] > mv
                mv = jnp.where(gt, cur[q], mv)
                pvl = jnp.where(gt, curp[q], pvl)
            sk, sv = plsc.sort_key_val(mv, pvl, descending=True)
            gmax = sk[0]
            gv = jnp.full((L,), gmax, jnp.float32)
            eq = mv == gv
            cnt = plsc.all_reduce_population_count(eq)[0]
            p = lax.cond(
                cnt > 1,
                lambda: _smin(jnp.where(eq, pvl, bigv)),
                lambda: sv[0])
            lane = j % L
            is_lo = j < L
            ins = iota == lane
            ov0 = jnp.where(jnp.logical_and(ins, is_lo), gmax, ov0)
            ov1 = jnp.where(jnp.logical_and(ins, ~is_lo), gmax, ov1)
            oi0 = jnp.where(jnp.logical_and(ins, is_lo), p, oi0)
            oi1 = jnp.where(jnp.logical_and(ins, ~is_lo), p, oi1)
            vb = (p // L) * L
            vv = rb[pl.ds(vb, L)]
            rb[pl.ds(vb, L)] = jnp.where(iota == p - vb, NEG, vv)
            g = p // GW
            dv = dbuf[pl.ds(g, L)]
            d = dv[0]
            dbuf[pl.ds(g, L)] = jnp.where(iota == 0, d + 1, dv)
            gbase = g * GW

            def from_table():
                return m2b[pl.ds(g, L)][0], p2b[pl.ds(g, L)][0]

            def rescan():
                def fa(t, mx):
                    return jnp.maximum(mx, rb[pl.ds(gbase + t * L, L)])

                nv = _smax(lax.fori_loop(0, GW // L, fa, negv))
                nvv = jnp.full((L,), nv, jnp.float32)

                def fbk(t, pv):
                    v = rb[pl.ds(gbase + t * L, L)]
                    return jnp.minimum(
                        pv,
                        jnp.where(v == nvv, gbase + t * L + iota, bigv))

                np_ = _smin(lax.fori_loop(0, GW // L, fbk, bigv))
                return nv, np_

            nv, np_ = lax.cond(d == 0, from_table, rescan)
            for q in range(NGV):
                eqg = iota == g - q * L
                cur[q] = jnp.where(eqg, nv, cur[q])
                curp[q] = jnp.where(eqg, np_, curp[q])
            return (ov0, ov1, oi0, oi1, *cur, *curp)

        def init_state(b, dbuf):
            _, mb, pb, _, _, _ = bufs[b]
            for q in range(NGV):
                dbuf[pl.ds(q * L, L)] = zerov
            cur = [mb[pl.ds(q * L, L)] for q in range(NGV)]
            curp = [pb[pl.ds(q * L, L)] for q in range(NGV)]
            return (negv, negv, bigv, bigv, *cur, *curp)

        def write_out(jrow, res):
            obase = jrow * K_TOP
            ovb[pl.ds(obase, L)] = res[0]
            ovb[pl.ds(obase + L, L)] = res[1]
            oib[pl.ds(obase, L)] = res[2]
            oib[pl.ds(obase + L, L)] = res[3]

        def process_pair(jrow_a, ba, bb):
            st_a = init_state(ba, depth0)
            st_b = init_state(bb, depth1)

            def jstep(j, carry):
                sa = carry[:12]
                sb = carry[12:]
                sa = one_step(j, sa, ba, depth0)
                sb = one_step(j, sb, bb, depth1)
                return (*sa, *sb)

            res = lax.fori_loop(0, K_TOP, jstep, (*st_a, *st_b))
            write_out(jrow_a, res[:12])
            write_out(jrow_a + 1, res[12:])

        def ring_rows(jj, carry):
            for bp in range(0, NB, 2):
                j = jj * NB + bp
                drain(base_row + j, bp)
                drain(base_row + j + 1, bp + 1)
                process_pair(j, bp, bp + 1)

                @pl.when(j + NB < RPW)
                def _():
                    fire(base_row + j + NB, bp)
                    fire(base_row + j + NB + 1, bp + 1)

            return carry

        lax.fori_loop(0, RPW // NB, ring_rows, 0)

        pltpu.sync_copy(ovb, vals_hbm.at[pl.ds(base_row * K_TOP, RPW * K_TOP)])
        pltpu.sync_copy(oib, idx_hbm.at[pl.ds(base_row * K_TOP, RPW * K_TOP)])

    vals_f, idx_f = run(sim, m1p, p1p, m2p, p2p)
    return vals_f.reshape(N, K_TOP), idx_f.reshape(N, K_TOP)


def kernel(fmap1, fmap2, k):
    B, C, H1, W1 = fmap1.shape
    H2, W2 = fmap2.shape[2], fmap2.shape[3]
    N, M = H1 * W1, H2 * W2
    f1 = fmap1.reshape(C, N)
    f2 = fmap2.reshape(C, M)

    sim, m1p, p1p, m2p, p2p = _sim_pallas(f1, f2, 256)
    vals, idx = _sc_topk(sim, m1p.reshape(-1), p1p.reshape(-1),
                         m2p.reshape(-1), p2p.reshape(-1))

    corr = (vals * (1.0 / jnp.sqrt(jnp.float32(C)))).T.reshape(B, K_TOP, N)
    idx_t = idx.T.reshape(B, K_TOP, N)

    m_idx = jnp.arange(M, dtype=jnp.int32)
    gy = (m_idx // W2).astype(jnp.float32)
    gx = (m_idx % W2).astype(jnp.float32)
    coords0 = jnp.broadcast_to(
        jnp.stack([gy, gx], axis=0)[None, :, None, :], (B, 2, K_TOP, M))
    cy = (idx_t // W2).astype(jnp.float32) - gy[:N]
    cx = (idx_t % W2).astype(jnp.float32) - gx[:N]
    coords1 = jnp.stack([cy, cx], axis=1)
    batch_index = jnp.zeros((B, 1, K_TOP, N), jnp.float32)
    corr = corr + (jnp.asarray(k) * 0).astype(corr.dtype)
    return (corr, coords0, coords1, batch_index)


# speedup vs baseline: 1.0004x; 1.0004x over previous
import functools

import jax
import jax.numpy as jnp
from jax import lax
from jax.experimental import pallas as pl
from jax.experimental.pallas import tpu as pltpu
from jax.experimental.pallas import tpu_sc as plsc

K_TOP = 32
L = 16
NC = 2
NS = 16
NW = NC * NS
GW = 128
GP = 64
NB = 6
NEG = float("-inf")
BIG = 1 << 30


def _matmul_kernel(f1_ref, f2_ref, sim_ref, m1_ref, p1_ref, m2_ref, p2_ref):
    s = jax.lax.dot_general(
        f1_ref[...], f2_ref[...], (((0,), (0,)), ((), ())),
        preferred_element_type=jnp.float32,
    )
    sim_ref[...] = s
    nt, m = s.shape
    ng = m // GW
    s3 = s.reshape(nt, ng, GW)
    gbase = jax.lax.broadcasted_iota(jnp.int32, (nt, ng), 1) * GW
    m1 = jnp.max(s3, axis=2)
    r1 = jnp.argmax(s3, axis=2).astype(jnp.int32)
    p1 = r1 + gbase
    colg = jax.lax.broadcasted_iota(jnp.int32, (nt, ng, GW), 2)
    s3b = jnp.where(colg == r1[..., None], NEG, s3)
    m2 = jnp.max(s3b, axis=2)
    p2 = jnp.argmax(s3b, axis=2).astype(jnp.int32) + gbase
    padv = jnp.full((nt, GP - ng), NEG, jnp.float32)
    padi = jnp.full((nt, GP - ng), BIG, jnp.int32)
    m1_ref[...] = jnp.concatenate([m1, padv], axis=1)
    p1_ref[...] = jnp.concatenate([p1, padi], axis=1)
    m2_ref[...] = jnp.concatenate([m2, padv], axis=1)
    p2_ref[...] = jnp.concatenate([p2, padi], axis=1)


def _sim_pallas(f1, f2, n_tile):
    C, N = f1.shape
    M = f2.shape[1]
    return pl.pallas_call(
        _matmul_kernel,
        grid=(N // n_tile,),
        in_specs=[
            pl.BlockSpec((C, n_tile), lambda i: (0, i)),
            pl.BlockSpec((C, M), lambda i: (0, 0)),
        ],
        out_specs=[
            pl.BlockSpec((n_tile, M), lambda i: (i, 0)),
            pl.BlockSpec((n_tile, GP), lambda i: (i, 0)),
            pl.BlockSpec((n_tile, GP), lambda i: (i, 0)),
            pl.BlockSpec((n_tile, GP), lambda i: (i, 0)),
            pl.BlockSpec((n_tile, GP), lambda i: (i, 0)),
        ],
        out_shape=[
            jax.ShapeDtypeStruct((N, M), jnp.float32),
            jax.ShapeDtypeStruct((N, GP), jnp.float32),
            jax.ShapeDtypeStruct((N, GP), jnp.int32),
            jax.ShapeDtypeStruct((N, GP), jnp.float32),
            jax.ShapeDtypeStruct((N, GP), jnp.int32),
        ],
    )(f1, f2)


def _smax(x):
    return lax.sort(x)[L - 1]


def _smin(x):
    return lax.sort(x)[0]


def _sc_topk(sim, m1p, p1p, m2p, p2p):
    N, M = sim.shape
    RPW = N // NW
    NG = M // GW
    NGV = GP // L

    mesh = plsc.VectorSubcoreMesh(core_axis_name="c", subcore_axis_name="s")

    aux = lambda: pltpu.VMEM((GP + L,), jnp.float32)
    auxi = lambda: pltpu.VMEM((GP + L,), jnp.int32)

    @functools.partial(
        pl.kernel, mesh=mesh,
        compiler_params=pltpu.CompilerParams(needs_layout_passes=False),
        out_type=[jax.ShapeDtypeStruct((N * K_TOP,), jnp.float32),
                  jax.ShapeDtypeStruct((N * K_TOP,), jnp.int32)],
        scratch_types=(
            [pltpu.VMEM((M,), jnp.float32) for _ in range(NB)]
            + [aux() for _ in range(NB)]
            + [auxi() for _ in range(NB)]
            + [aux() for _ in range(NB)]
            + [auxi() for _ in range(NB)]
            + [auxi(), auxi(),
               pltpu.VMEM((RPW * K_TOP,), jnp.float32),
               pltpu.VMEM((RPW * K_TOP,), jnp.int32)]
            + [pltpu.SemaphoreType.DMA for _ in range(NB)]
        ),
    )
    def run(sim_hbm, m1_hbm, p1_hbm, m2_hbm, p2_hbm, vals_hbm, idx_hbm,
            *scr):
        rbs = scr[0:NB]
        mbs = scr[NB:2 * NB]
        pbs = scr[2 * NB:3 * NB]
        m2bs = scr[3 * NB:4 * NB]
        p2bs = scr[4 * NB:5 * NB]
        depth0, depth1, ovb, oib = scr[5 * NB:5 * NB + 4]
        sems = scr[5 * NB + 4:5 * NB + 4 + NB]
        wid = lax.axis_index("s") * NC + lax.axis_index("c")
        base_row = wid * RPW
        iota = lax.iota(jnp.int32, L)
        negv = jnp.full((L,), NEG, jnp.float32)
        bigv = jnp.full((L,), BIG, jnp.int32)
        zerov = jnp.full((L,), 0, jnp.int32)

        bufs = tuple(
            (rbs[b], mbs[b], pbs[b], m2bs[b], p2bs[b], sems[b])
            for b in range(NB))

        def fire(row, b):
            rb, mb, pb, m2b, p2b, sem = bufs[b]
            pltpu.async_copy(sim_hbm.at[row], rb, sem)
            pltpu.async_copy(m1_hbm.at[pl.ds(row * GP, GP)], mb.at[pl.ds(0, GP)], sem)
            pltpu.async_copy(p1_hbm.at[pl.ds(row * GP, GP)], pb.at[pl.ds(0, GP)], sem)
            pltpu.async_copy(m2_hbm.at[pl.ds(row * GP, GP)], m2b.at[pl.ds(0, GP)], sem)
            pltpu.async_copy(p2_hbm.at[pl.ds(row * GP, GP)], p2b.at[pl.ds(0, GP)], sem)

        def drain(row, b):
            rb, mb, pb, m2b, p2b, sem = bufs[b]
            pltpu.make_async_copy(sim_hbm.at[row], rb, sem).wait()
            pltpu.make_async_copy(m1_hbm.at[pl.ds(row * GP, GP)], mb.at[pl.ds(0, GP)], sem).wait()
            pltpu.make_async_copy(p1_hbm.at[pl.ds(row * GP, GP)], pb.at[pl.ds(0, GP)], sem).wait()
            pltpu.make_async_copy(m2_hbm.at[pl.ds(row * GP, GP)], m2b.at[pl.ds(0, GP)], sem).wait()
            pltpu.make_async_copy(p2_hbm.at[pl.ds(row * GP, GP)], p2b.at[pl.ds(0, GP)], sem).wait()

        for b in range(NB):
            fire(base_row + b, b)

        def one_step(j, st, b, dbuf):
            rb, mb, pb, m2b, p2b, _ = bufs[b]
            ov0, ov1, oi0, oi1, c0, c1, c2, c3, q0, q1, q2, q3 = st
            cur = [c0, c1, c2, c3]
            curp = [q0, q1, q2, q3]
            mv, pvl = cur[0], curp[0]
            for q in range(1, NGV):
                gt = cur[q] > mv
                mv = jnp.where(gt, cur[q], mv)
                pvl = jnp.where(gt, curp[q], pvl)
            sk, sv = plsc.sort_key_val(mv, pvl, descending=True)
            gmax = sk[0]
            gv = jnp.full((L,), gmax, jnp.float32)
            eq = mv == gv
            cnt = plsc.all_reduce_population_count(eq)[0]
            p = lax.cond(
                cnt > 1,
                lambda: _smin(jnp.where(eq, pvl, bigv)),
                lambda: sv[0])
            lane = j % L
            is_lo = j < L
            ins = iota == lane
            ov0 = jnp.where(jnp.logical_and(ins, is_lo), gmax, ov0)
            ov1 = jnp.where(jnp.logical_and(ins, ~is_lo), gmax, ov1)
            oi0 = jnp.where(jnp.logical_and(ins, is_lo), p, oi0)
            oi1 = jnp.where(jnp.logical_and(ins, ~is_lo), p, oi1)
            vb = (p // L) * L
            vv = rb[pl.ds(vb, L)]
            rb[pl.ds(vb, L)] = jnp.where(iota == p - vb, NEG, vv)
            g = p // GW
            dv = dbuf[pl.ds(g, L)]
            d = dv[0]
            dbuf[pl.ds(g, L)] = jnp.where(iota == 0, d + 1, dv)
            gbase = g * GW

            def from_table():
                return m2b[pl.ds(g, L)][0], p2b[pl.ds(g, L)][0]

            def rescan():
                def fa(t, mx):
                    return jnp.maximum(mx, rb[pl.ds(gbase + t * L, L)])

                nv = _smax(lax.fori_loop(0, GW // L, fa, negv))
                nvv = jnp.full((L,), nv, jnp.float32)

                def fbk(t, pv):
                    v = rb[pl.ds(gbase + t * L, L)]
                    return jnp.minimum(
                        pv,
                        jnp.where(v == nvv, gbase + t * L + iota, bigv))

                np_ = _smin(lax.fori_loop(0, GW // L, fbk, bigv))
                return nv, np_

            nv, np_ = lax.cond(d == 0, from_table, rescan)
            for q in range(NGV):
                eqg = iota == g - q * L
                cur[q] = jnp.where(eqg, nv, cur[q])
                curp[q] = jnp.where(eqg, np_, curp[q])
            return (ov0, ov1, oi0, oi1, *cur, *curp)

        def init_state(b, dbuf):
            _, mb, pb, _, _, _ = bufs[b]
            for q in range(NGV):
                dbuf[pl.ds(q * L, L)] = zerov
            cur = [mb[pl.ds(q * L, L)] for q in range(NGV)]
            curp = [pb[pl.ds(q * L, L)] for q in range(NGV)]
            return (negv, negv, bigv, bigv, *cur, *curp)

        def write_out(jrow, res):
            obase = jrow * K_TOP
            ovb[pl.ds(obase, L)] = res[0]
            ovb[pl.ds(obase + L, L)] = res[1]
            oib[pl.ds(obase, L)] = res[2]
            oib[pl.ds(obase + L, L)] = res[3]

        def process_pair(jrow_a, ba, bb):
            st_a = init_state(ba, depth0)
            st_b = init_state(bb, depth1)

            def jstep(j, carry):
                sa = carry[:12]
                sb = carry[12:]
                sa = one_step(j, sa, ba, depth0)
                sb = one_step(j, sb, bb, depth1)
                return (*sa, *sb)

            res = lax.fori_loop(0, K_TOP, jstep, (*st_a, *st_b))
            write_out(jrow_a, res[:12])
            write_out(jrow_a + 1, res[12:])

        def ring_rows(jj, carry):
            for bp in range(0, NB, 2):
                j = jj * NB + bp
                drain(base_row + j, bp)
                drain(base_row + j + 1, bp + 1)
                process_pair(j, bp, bp + 1)

                @pl.when(j + NB < RPW)
                def _():
                    fire(base_row + j + NB, bp)
                    fire(base_row + j + NB + 1, bp + 1)

            return carry

        lax.fori_loop(0, RPW // NB, ring_rows, 0)

        pltpu.sync_copy(ovb, vals_hbm.at[pl.ds(base_row * K_TOP, RPW * K_TOP)])
        pltpu.sync_copy(oib, idx_hbm.at[pl.ds(base_row * K_TOP, RPW * K_TOP)])

    vals_f, idx_f = run(sim, m1p, p1p, m2p, p2p)
    return vals_f.reshape(N, K_TOP), idx_f.reshape(N, K_TOP)


def kernel(fmap1, fmap2, k):
    B, C, H1, W1 = fmap1.shape
    H2, W2 = fmap2.shape[2], fmap2.shape[3]
    N, M = H1 * W1, H2 * W2
    f1 = fmap1.reshape(C, N)
    f2 = fmap2.reshape(C, M)

    sim, m1p, p1p, m2p, p2p = _sim_pallas(f1, f2, 256)
    vals, idx = _sc_topk(sim, m1p.reshape(-1), p1p.reshape(-1),
                         m2p.reshape(-1), p2p.reshape(-1))

    corr = (vals * (1.0 / jnp.sqrt(jnp.float32(C)))).T.reshape(B, K_TOP, N)
    idx_t = idx.T.reshape(B, K_TOP, N)

    m_idx = jnp.arange(M, dtype=jnp.int32)
    gy = (m_idx // W2).astype(jnp.float32)
    gx = (m_idx % W2).astype(jnp.float32)
    coords0 = jnp.broadcast_to(
        jnp.stack([gy, gx], axis=0)[None, :, None, :], (B, 2, K_TOP, M))
    cy = (idx_t // W2).astype(jnp.float32) - gy[:N]
    cx = (idx_t % W2).astype(jnp.float32) - gx[:N]
    coords1 = jnp.stack([cy, cx], axis=1)
    batch_index = jnp.zeros((B, 1, K_TOP, N), jnp.float32)
    corr = corr + (jnp.asarray(k) * 0).astype(corr.dtype)
    return (corr, coords0, coords1, batch_index)
